# baseline (speedup 1.0000x reference)
import functools
import jax
import jax.numpy as jnp
from jax import lax
from jax.experimental import pallas as pl
from jax.experimental.pallas import tpu as pltpu, tpu_sc as plsc


def _make_sc_gather(B, V, D):
    info = plsc.get_sparse_core_info()
    L = info.num_lanes
    NC = info.num_cores
    NW = NC * info.num_subcores
    assert B % (8 * NW) == 0 and 128 % D == 0 and V % (128 // D) == 0
    R = 128 // D
    b_per_w = B // NW
    CHUNK = 128
    assert b_per_w % CHUNK == 0

    mesh = plsc.VectorSubcoreMesh(core_axis_name="c", subcore_axis_name="s")

    @functools.partial(
        pl.kernel,
        mesh=mesh,
        out_type=jax.ShapeDtypeStruct((B, D), jnp.float32),
        compiler_params=pltpu.CompilerParams(needs_layout_passes=False),
        scratch_types=[
            pltpu.VMEM((b_per_w,), jnp.int32),
            pltpu.VMEM((b_per_w,), jnp.int32),
            pltpu.VMEM((CHUNK, 128), jnp.float32),
            pltpu.VMEM((b_per_w, D), jnp.float32),
            pltpu.SemaphoreType.DMA,
        ],
    )
    def emb(labels_hbm, table_hbm, out_hbm, idx_v, big_v, rows_v, out_v, sem):
        wid = lax.axis_index("s") * NC + lax.axis_index("c")
        base = wid * b_per_w
        pltpu.sync_copy(labels_hbm.at[pl.ds(base, b_per_w)], idx_v)

        lane = lax.iota(jnp.int32, L)

        def to_big(k, _):
            v = idx_v[pl.ds(k * L, L)]
            big_v[pl.ds(k * L, L)] = lax.shift_right_logical(v, R.bit_length() - 1)
            return _

        lax.fori_loop(0, b_per_w // L, to_big, 0)

        def one_pass(p, _):
            pltpu.async_copy(
                table_hbm.at[big_v.at[pl.ds(p * CHUNK, CHUNK)]], rows_v, sem
            ).wait()

            def extract(k, _):
                jvec = k * L + lane
                labv = idx_v[pl.ds(p * CHUNK + k * L, L)]
                colbase = (labv & (R - 1)) * D
                for c in range(D):
                    vals = plsc.load_gather(rows_v, [jvec, colbase + c])
                    plsc.store_scatter(
                        out_v, [p * CHUNK + jvec, lane * 0 + c], vals
                    )
                return _

            lax.fori_loop(0, CHUNK // L, extract, 0)
            return _

        lax.fori_loop(0, b_per_w // CHUNK, one_pass, 0)
        pltpu.sync_copy(out_v, out_hbm.at[pl.ds(base, b_per_w)])

    return emb


def kernel(labels, table):
    B = labels.shape[0]
    V, D = table.shape
    emb = _make_sc_gather(B, V, D)
    tbig = table.reshape(V * D // 128, 128)
    return emb(labels.astype(jnp.int32), tbig)


# speedup vs baseline: 3.9146x; 3.9146x over previous
import functools
import jax
import jax.numpy as jnp
from jax import lax
from jax.experimental import pallas as pl
from jax.experimental.pallas import tpu as pltpu, tpu_sc as plsc


def _make_sc_gather(B, V, D):
    info = plsc.get_sparse_core_info()
    NC = info.num_cores
    NW = NC * info.num_subcores
    assert B % (16 * NW) == 0 and D % 16 == 0
    b_per_w = B // NW
    G = 8
    NIT = b_per_w // 16
    LAST = (V // 128) * 128
    TAILW = V - LAST
    mesh = plsc.VectorSubcoreMesh(core_axis_name="c", subcore_axis_name="s")

    @functools.partial(
        pl.kernel,
        mesh=mesh,
        out_type=jax.ShapeDtypeStruct((D, B), jnp.float32),
        compiler_params=pltpu.CompilerParams(needs_layout_passes=False),
        scratch_types=[
            pltpu.VMEM((b_per_w,), jnp.int32),
            pltpu.VMEM((2, G, D, 128), jnp.float32),
            pltpu.VMEM((D, b_per_w), jnp.float32),
            pltpu.VMEM((D, max(V - (V // 128) * 128, 1)), jnp.float32),
            pltpu.SemaphoreType.DMA,
            pltpu.SemaphoreType.DMA,
        ],
    )
    def emb(labels_hbm, tableT_hbm, outT_hbm,
            idx_v, ring_v, out_v, tail_v, sem0, sem1):
        wid = lax.axis_index("s") * NC + lax.axis_index("c")
        base = wid * b_per_w
        pltpu.sync_copy(labels_hbm.at[pl.ds(base, b_per_w)], idx_v)

        lane16 = lax.iota(jnp.int32, 16)

        def issue(vec, off, buf, sem):
            for l in range(G):
                i = vec[off + l]
                col0 = jnp.where(i >= LAST, LAST - 128, i & -128)
                col0 = pl.multiple_of(col0, 128)
                pltpu.async_copy(
                    tableT_hbm.at[:, pl.ds(col0, 128)],
                    ring_v.at[buf, l],
                    sem,
                )

        def drain(buf, sem):
            for l in range(G):
                pltpu.make_async_copy(
                    tableT_hbm.at[:, pl.ds(0, 128)], ring_v.at[buf, l], sem
                ).wait()

        def extract(vec, off, buf, slot0):
            for l in range(G):
                i = vec[off + l]
                lane = i & 127
                slot = slot0 + l
                for h in range(D // 16):
                    vals = plsc.load_gather(
                        ring_v.at[buf, l],
                        [h * 16 + lane16, lane16 * 0 + lane],
                    )
                    plsc.store_scatter(
                        out_v,
                        [h * 16 + lane16, lane16 * 0 + slot],
                        vals,
                    )

        vec0 = idx_v[pl.ds(0, 16)]
        issue(vec0, 0, 0, sem0)

        def step(it, c):
            vec = idx_v[pl.ds(it * 16, 16)]
            issue(vec, G, 1, sem1)
            drain(0, sem0)
            extract(vec, 0, 0, it * 16)

            @pl.when(it + 1 < NIT)
            def _():
                vecn = idx_v[pl.ds((it + 1) * 16, 16)]
                issue(vecn, 0, 0, sem0)

            drain(1, sem1)
            extract(vec, G, 1, it * 16 + G)
            return c

        lax.fori_loop(0, NIT, step, 0)

        if TAILW > 0:
            def fix_tail(k, c):
                vec = idx_v[pl.ds(k * 16, 16)]

                @pl.when(jnp.any(vec >= LAST))
                def _():
                    for l in range(16):
                        i = vec[l]

                        @pl.when(i >= LAST)
                        def _():
                            pltpu.async_copy(
                                tableT_hbm.at[:, pl.ds(LAST, TAILW)],
                                tail_v,
                                sem0,
                            ).wait()
                            lane = i - LAST
                            slot = k * 16 + l
                            for h in range(D // 16):
                                vals = plsc.load_gather(
                                    tail_v,
                                    [h * 16 + lane16, lane16 * 0 + lane],
                                )
                                plsc.store_scatter(
                                    out_v,
                                    [h * 16 + lane16, lane16 * 0 + slot],
                                    vals,
                                )

                return c

            lax.fori_loop(0, NIT, fix_tail, 0)

        pltpu.sync_copy(out_v, outT_hbm.at[:, pl.ds(base, b_per_w)])

    return emb


def kernel(labels, table):
    B = labels.shape[0]
    V, D = table.shape
    emb = _make_sc_gather(B, V, D)
    outT = emb(labels.astype(jnp.int32), table.T)
    return outT.T


# speedup vs baseline: 4.2541x; 1.0867x over previous
import functools
import jax
import jax.numpy as jnp
from jax import lax
from jax.experimental import pallas as pl
from jax.experimental.pallas import tpu as pltpu, tpu_sc as plsc


def _make_sc_gather(B, V, D):
    info = plsc.get_sparse_core_info()
    NC = info.num_cores
    NW = NC * info.num_subcores
    assert B % (16 * NW) == 0 and D % 16 == 0
    b_per_w = B // NW
    G = 8
    NIT = b_per_w // 16
    LAST = (V // 128) * 128
    TAILW = V - LAST
    mesh = plsc.VectorSubcoreMesh(core_axis_name="c", subcore_axis_name="s")

    @functools.partial(
        pl.kernel,
        mesh=mesh,
        out_type=jax.ShapeDtypeStruct((D, B), jnp.float32),
        compiler_params=pltpu.CompilerParams(needs_layout_passes=False),
        scratch_types=[
            pltpu.VMEM((b_per_w,), jnp.int32),
            pltpu.VMEM((3, G, D, 128), jnp.float32),
            pltpu.VMEM((D, b_per_w), jnp.float32),
            pltpu.VMEM((D, max(V - (V // 128) * 128, 1)), jnp.float32),
            pltpu.SemaphoreType.DMA,
            pltpu.SemaphoreType.DMA,
            pltpu.SemaphoreType.DMA,
        ],
    )
    def emb(labels_hbm, tableT_hbm, outT_hbm,
            idx_v, ring_v, out_v, tail_v, sem0, sem1, sem2):
        wid = lax.axis_index("s") * NC + lax.axis_index("c")
        base = wid * b_per_w
        pltpu.sync_copy(labels_hbm.at[pl.ds(base, b_per_w)], idx_v)

        lane16 = lax.iota(jnp.int32, 16)
        sems = (sem0, sem1, sem2)
        NG = b_per_w // G

        def get_i(g, l):
            vec = idx_v[pl.ds((g // 2) * 16, 16)]
            return jnp.where(g % 2 == 0, vec[l], vec[G + l])

        def issue(g, buf):
            for l in range(G):
                i = get_i(g, l)
                col0 = jnp.where(i >= LAST, LAST - 128, i & -128)
                col0 = pl.multiple_of(col0, 128)
                pltpu.async_copy(
                    tableT_hbm.at[:, pl.ds(col0, 128)],
                    ring_v.at[buf, l],
                    sems[buf],
                )

        def drain(buf):
            for l in range(G):
                pltpu.make_async_copy(
                    tableT_hbm.at[:, pl.ds(0, 128)], ring_v.at[buf, l], sems[buf]
                ).wait()

        def extract(g, buf):
            for l in range(G):
                i = get_i(g, l)
                lane = i & 127
                slot = g * G + l
                for h in range(D // 16):
                    vals = plsc.load_gather(
                        ring_v.at[buf, l],
                        [h * 16 + lane16, lane16 * 0 + lane],
                    )
                    plsc.store_scatter(
                        out_v,
                        [h * 16 + lane16, lane16 * 0 + slot],
                        vals,
                    )

        issue(0, 0)
        issue(1, 1)
        NBODY = (NG - 1) // 3

        def step(it, c):
            for sub in range(3):
                g = 3 * it + sub

                @pl.when(g + 2 < NG)
                def _():
                    issue(g + 2, (sub + 2) % 3)

                drain(sub)
                extract(g, sub)
            return c

        lax.fori_loop(0, NBODY, step, 0)
        for g in range(3 * NBODY, NG):
            drain(g % 3)
            extract(g, g % 3)

        if TAILW > 0:
            def fix_tail(k, c):
                vec = idx_v[pl.ds(k * 16, 16)]

                @pl.when(jnp.any(vec >= LAST))
                def _():
                    for l in range(16):
                        i = vec[l]

                        @pl.when(i >= LAST)
                        def _():
                            pltpu.async_copy(
                                tableT_hbm.at[:, pl.ds(LAST, TAILW)],
                                tail_v,
                                sem0,
                            ).wait()
                            lane = i - LAST
                            slot = k * 16 + l
                            for h in range(D // 16):
                                vals = plsc.load_gather(
                                    tail_v,
                                    [h * 16 + lane16, lane16 * 0 + lane],
                                )
                                plsc.store_scatter(
                                    out_v,
                                    [h * 16 + lane16, lane16 * 0 + slot],
                                    vals,
                                )

                return c

            lax.fori_loop(0, NIT, fix_tail, 0)

        pltpu.sync_copy(out_v, outT_hbm.at[:, pl.ds(base, b_per_w)])

    return emb


def kernel(labels, table):
    B = labels.shape[0]
    V, D = table.shape
    emb = _make_sc_gather(B, V, D)
    outT = emb(labels.astype(jnp.int32), table.T)
    return outT.T
